# baseline (speedup 1.0000x reference)
import jax
import jax.numpy as jnp
from jax.experimental import pallas as pl
from jax.experimental.pallas import tpu as pltpu

_M_BLK = 200
_C_BLKS = 8


def _proj_kernel(x_ref, w1a_ref, w1b_ref, sa_ref, sb_ref):
    sa_ref[...] = jnp.dot(x_ref[...], w1a_ref[...],
                          preferred_element_type=jnp.float32
                          ).astype(jnp.bfloat16)
    sb_ref[...] = jnp.dot(x_ref[...], w1b_ref[...],
                          preferred_element_type=jnp.float32
                          ).astype(jnp.bfloat16)


def _branch_kernel(adj_ref, s_ref, b1_ref, w2_ref, wp_ref, b2_ref,
                   x_ref, p_ref, t_ref, cache_ref):
    ph = pl.program_id(0)
    i = pl.program_id(1)

    def _layer2(adj16):
        x = jnp.dot(adj16, t_ref[...].astype(jnp.bfloat16),
                    preferred_element_type=jnp.float32) + b2_ref[...]
        x_ref[...] = x
        p_ref[...] = jnp.maximum(
            jnp.dot(jnp.maximum(x, 0.0), wp_ref[...],
                    preferred_element_type=jnp.float32), 0.0)

    @pl.when(ph == 0)
    def _():
        adj16 = adj_ref[...].astype(jnp.bfloat16)
        h = jnp.maximum(
            jnp.dot(adj16, s_ref[...],
                    preferred_element_type=jnp.float32) + b1_ref[...], 0.0)
        t_ref[pl.ds(i * _M_BLK, _M_BLK), :] = jnp.dot(
            h, w2_ref[...], preferred_element_type=jnp.float32)

        @pl.when(i < _C_BLKS)
        def _():
            cache_ref[jnp.minimum(i, _C_BLKS - 1)] = adj16

    @pl.when(jnp.logical_and(ph == 1, i < _C_BLKS))
    def _():
        _layer2(cache_ref[jnp.minimum(i, _C_BLKS - 1)])

    @pl.when(jnp.logical_and(ph == 1, i >= _C_BLKS))
    def _():
        _layer2(adj_ref[...].astype(jnp.bfloat16))


def _branch(adj, s, b1, w2, b2, wp):
    n = adj.shape[0]
    nh1 = s.shape[1]
    nh2 = w2.shape[1]
    ncls = wp.shape[1]
    nblk = n // _M_BLK
    grid = (2, nblk)
    _full = lambda shape: pl.BlockSpec(shape, lambda p, i: (0, 0))

    def _adj_map(p, i):
        cached = jnp.logical_and(p == 1, i < _C_BLKS)
        return (jnp.where(cached, nblk - 1, i), 0)

    _out = lambda w: pl.BlockSpec((_M_BLK, w), lambda p, i: (i * p, 0))

    x, p = pl.pallas_call(
        _branch_kernel,
        grid=grid,
        in_specs=[
            pl.BlockSpec((_M_BLK, n), _adj_map),
            _full((n, nh1)), _full((1, nh1)), _full((nh1, nh2)),
            _full((nh2, ncls)), _full((1, nh2)),
        ],
        out_specs=[_out(nh2), _out(ncls)],
        out_shape=[
            jax.ShapeDtypeStruct((n, nh2), jnp.float32),
            jax.ShapeDtypeStruct((n, ncls), jnp.float32),
        ],
        scratch_shapes=[
            pltpu.VMEM((n, nh2), jnp.float32),
            pltpu.VMEM((_C_BLKS, _M_BLK, n), jnp.bfloat16),
        ],
        compiler_params=pltpu.CompilerParams(
            dimension_semantics=("arbitrary", "arbitrary"),
            vmem_limit_bytes=63 * 1024 * 1024,
        ),
    )(adj, s, b1.reshape(1, -1), w2, wp, b2.reshape(1, -1))
    return x, p


def kernel(X, nsadj, nfadj, W1a, b1a, W2a, b2a, W1b, b1b, W2b, b2b, Wp):
    n, nfeat = X.shape
    nh1 = W1a.shape[1]

    sa, sb = pl.pallas_call(
        _proj_kernel,
        out_shape=[
            jax.ShapeDtypeStruct((n, nh1), jnp.bfloat16),
            jax.ShapeDtypeStruct((n, nh1), jnp.bfloat16),
        ],
    )(X, W1a, W1b)

    x1, p1 = _branch(nsadj, sa, b1a, W2a, b2a, Wp)
    x2, p2 = _branch(nfadj, sb, b1b, W2b, b2b, Wp)
    return (p1, p2, x1, x2)


# speedup vs baseline: 1.0023x; 1.0023x over previous
import jax
import jax.numpy as jnp
from jax.experimental import pallas as pl
from jax.experimental.pallas import tpu as pltpu

_M_BLK = 200
_N_BUF = 3


def _adj_spec(n):
    return pl.BlockSpec((_M_BLK, n), lambda i: (i, 0),
                        pipeline_mode=pl.Buffered(buffer_count=_N_BUF))


def _proj_kernel(x_ref, w1a_ref, w1b_ref, sa_ref, sb_ref):
    sa_ref[...] = jnp.dot(x_ref[...], w1a_ref[...],
                          preferred_element_type=jnp.float32
                          ).astype(jnp.bfloat16)
    sb_ref[...] = jnp.dot(x_ref[...], w1b_ref[...],
                          preferred_element_type=jnp.float32
                          ).astype(jnp.bfloat16)


def _k1(nsadj_hbm, nfadj_hbm, sa_ref, sb_ref, b1a_ref, w2a_ref,
        b1b_ref, w2b_ref, ta_hbm, tb_hbm):
    n = nsadj_hbm.shape[0]
    nh2 = w2a_ref.shape[1]

    def body(ns_ref, nf_ref, ta_ref, tb_ref):
        ha = jnp.maximum(
            jnp.dot(ns_ref[...].astype(jnp.bfloat16), sa_ref[...],
                    preferred_element_type=jnp.float32) + b1a_ref[...], 0.0)
        ta_ref[...] = jnp.dot(
            ha, w2a_ref[...],
            preferred_element_type=jnp.float32).astype(jnp.bfloat16)
        hb = jnp.maximum(
            jnp.dot(nf_ref[...].astype(jnp.bfloat16), sb_ref[...],
                    preferred_element_type=jnp.float32) + b1b_ref[...], 0.0)
        tb_ref[...] = jnp.dot(
            hb, w2b_ref[...],
            preferred_element_type=jnp.float32).astype(jnp.bfloat16)

    pltpu.emit_pipeline(
        body,
        grid=(n // _M_BLK,),
        in_specs=[_adj_spec(n), _adj_spec(n)],
        out_specs=[
            pl.BlockSpec((_M_BLK, nh2), lambda i: (i, 0)),
            pl.BlockSpec((_M_BLK, nh2), lambda i: (i, 0)),
        ],
    )(nsadj_hbm, nfadj_hbm, ta_hbm, tb_hbm)


def _k2(nsadj_hbm, nfadj_hbm, ta_ref, tb_ref, b2a_ref, b2b_ref, wp_ref,
        x1_hbm, x2_hbm, p1_hbm, p2_hbm):
    n = ta_ref.shape[0]
    nh2 = ta_ref.shape[1]
    ncls = wp_ref.shape[1]

    def body(ns_ref, nf_ref, x1_ref, x2_ref, p1_ref, p2_ref):
        x1 = jnp.dot(ns_ref[...].astype(jnp.bfloat16), ta_ref[...],
                     preferred_element_type=jnp.float32) + b2a_ref[...]
        x1_ref[...] = x1
        p1_ref[...] = jnp.maximum(
            jnp.dot(jnp.maximum(x1, 0.0), wp_ref[...],
                    preferred_element_type=jnp.float32), 0.0)
        x2 = jnp.dot(nf_ref[...].astype(jnp.bfloat16), tb_ref[...],
                     preferred_element_type=jnp.float32) + b2b_ref[...]
        x2_ref[...] = x2
        p2_ref[...] = jnp.maximum(
            jnp.dot(jnp.maximum(x2, 0.0), wp_ref[...],
                    preferred_element_type=jnp.float32), 0.0)

    pltpu.emit_pipeline(
        body,
        grid=(n // _M_BLK,),
        in_specs=[_adj_spec(n), _adj_spec(n)],
        out_specs=[
            pl.BlockSpec((_M_BLK, nh2), lambda i: (i, 0)),
            pl.BlockSpec((_M_BLK, nh2), lambda i: (i, 0)),
            pl.BlockSpec((_M_BLK, ncls), lambda i: (i, 0)),
            pl.BlockSpec((_M_BLK, ncls), lambda i: (i, 0)),
        ],
    )(nsadj_hbm, nfadj_hbm, x1_hbm, x2_hbm, p1_hbm, p2_hbm)


def kernel(X, nsadj, nfadj, W1a, b1a, W2a, b2a, W1b, b1b, W2b, b2b, Wp):
    n, nfeat = X.shape
    nh1 = W1a.shape[1]
    nh2 = W2a.shape[1]
    ncls = Wp.shape[1]

    _any = pl.BlockSpec(memory_space=pl.ANY)
    _vmem = pl.BlockSpec(memory_space=pltpu.MemorySpace.VMEM)

    sa, sb = pl.pallas_call(
        _proj_kernel,
        out_shape=[
            jax.ShapeDtypeStruct((n, nh1), jnp.bfloat16),
            jax.ShapeDtypeStruct((n, nh1), jnp.bfloat16),
        ],
    )(X, W1a, W1b)

    ta, tb = pl.pallas_call(
        _k1,
        in_specs=[_any, _any] + [_vmem] * 6,
        out_specs=[_any, _any],
        out_shape=[
            jax.ShapeDtypeStruct((n, nh2), jnp.bfloat16),
            jax.ShapeDtypeStruct((n, nh2), jnp.bfloat16),
        ],
        compiler_params=pltpu.CompilerParams(
            vmem_limit_bytes=63 * 1024 * 1024,
        ),
    )(nsadj, nfadj, sa, sb, b1a.reshape(1, -1), W2a,
      b1b.reshape(1, -1), W2b)

    x1, x2, p1, p2 = pl.pallas_call(
        _k2,
        in_specs=[_any, _any] + [_vmem] * 5,
        out_specs=[_any, _any, _any, _any],
        out_shape=[
            jax.ShapeDtypeStruct((n, nh2), jnp.float32),
            jax.ShapeDtypeStruct((n, nh2), jnp.float32),
            jax.ShapeDtypeStruct((n, ncls), jnp.float32),
            jax.ShapeDtypeStruct((n, ncls), jnp.float32),
        ],
        compiler_params=pltpu.CompilerParams(
            vmem_limit_bytes=63 * 1024 * 1024,
        ),
    )(nsadj, nfadj, ta, tb, b2a.reshape(1, -1), b2b.reshape(1, -1), Wp)

    return (p1, p2, x1, x2)


# speedup vs baseline: 1.0219x; 1.0196x over previous
import jax
import jax.numpy as jnp
from jax.experimental import pallas as pl
from jax.experimental.pallas import tpu as pltpu

_M_BLK = 200


def _k1(nsadj_ref, nfadj_ref, x_ref, w1a_ref, b1a_ref, w2a_ref,
        w1b_ref, b1b_ref, w2b_ref, ta_ref, tb_ref, sa_ref, sb_ref):
    i = pl.program_id(0)

    @pl.when(i == 0)
    def _():
        sa_ref[...] = jnp.dot(x_ref[...], w1a_ref[...],
                              preferred_element_type=jnp.float32
                              ).astype(jnp.bfloat16)
        sb_ref[...] = jnp.dot(x_ref[...], w1b_ref[...],
                              preferred_element_type=jnp.float32
                              ).astype(jnp.bfloat16)

    ha = jnp.maximum(
        jnp.dot(nsadj_ref[...].astype(jnp.bfloat16), sa_ref[...],
                preferred_element_type=jnp.float32) + b1a_ref[...], 0.0)
    ta_ref[...] = jnp.dot(ha, w2a_ref[...], preferred_element_type=jnp.float32)
    hb = jnp.maximum(
        jnp.dot(nfadj_ref[...].astype(jnp.bfloat16), sb_ref[...],
                preferred_element_type=jnp.float32) + b1b_ref[...], 0.0)
    tb_ref[...] = jnp.dot(hb, w2b_ref[...], preferred_element_type=jnp.float32)


def _k2(nsadj_ref, nfadj_ref, ta_ref, tb_ref, b2a_ref, b2b_ref, wp_ref,
        x1_ref, x2_ref, p1_ref, p2_ref):
    x1 = jnp.dot(nsadj_ref[...].astype(jnp.bfloat16),
                 ta_ref[...].astype(jnp.bfloat16),
                 preferred_element_type=jnp.float32) + b2a_ref[...]
    x1_ref[...] = x1
    p1_ref[...] = jnp.maximum(
        jnp.dot(jnp.maximum(x1, 0.0), wp_ref[...],
                preferred_element_type=jnp.float32), 0.0)
    x2 = jnp.dot(nfadj_ref[...].astype(jnp.bfloat16),
                 tb_ref[...].astype(jnp.bfloat16),
                 preferred_element_type=jnp.float32) + b2b_ref[...]
    x2_ref[...] = x2
    p2_ref[...] = jnp.maximum(
        jnp.dot(jnp.maximum(x2, 0.0), wp_ref[...],
                preferred_element_type=jnp.float32), 0.0)


def kernel(X, nsadj, nfadj, W1a, b1a, W2a, b2a, W1b, b1b, W2b, b2b, Wp):
    n, nfeat = X.shape
    nh1 = W1a.shape[1]
    nh2 = W2a.shape[1]
    ncls = Wp.shape[1]
    grid = (n // _M_BLK,)

    _full = lambda shape: pl.BlockSpec(shape, lambda i: (0, 0))
    _rows = lambda w: pl.BlockSpec((_M_BLK, w), lambda i: (i, 0))

    ta, tb = pl.pallas_call(
        _k1,
        grid=grid,
        in_specs=[
            _rows(n), _rows(n),
            _full((n, nfeat)),
            _full((nfeat, nh1)), _full((1, nh1)), _full((nh1, nh2)),
            _full((nfeat, nh1)), _full((1, nh1)), _full((nh1, nh2)),
        ],
        out_specs=[_rows(nh2), _rows(nh2)],
        out_shape=[
            jax.ShapeDtypeStruct((n, nh2), jnp.float32),
            jax.ShapeDtypeStruct((n, nh2), jnp.float32),
        ],
        scratch_shapes=[
            pltpu.VMEM((n, nh1), jnp.bfloat16),
            pltpu.VMEM((n, nh1), jnp.bfloat16),
        ],
        compiler_params=pltpu.CompilerParams(
            dimension_semantics=("arbitrary",),
        ),
    )(nsadj, nfadj, X, W1a, b1a.reshape(1, -1), W2a,
      W1b, b1b.reshape(1, -1), W2b)

    x1, x2, p1, p2 = pl.pallas_call(
        _k2,
        grid=grid,
        in_specs=[
            _rows(n), _rows(n),
            _full((n, nh2)), _full((n, nh2)),
            _full((1, nh2)), _full((1, nh2)),
            _full((nh2, ncls)),
        ],
        out_specs=[_rows(nh2), _rows(nh2), _rows(ncls), _rows(ncls)],
        out_shape=[
            jax.ShapeDtypeStruct((n, nh2), jnp.float32),
            jax.ShapeDtypeStruct((n, nh2), jnp.float32),
            jax.ShapeDtypeStruct((n, ncls), jnp.float32),
            jax.ShapeDtypeStruct((n, ncls), jnp.float32),
        ],
        compiler_params=pltpu.CompilerParams(
            dimension_semantics=("arbitrary",),
        ),
    )(nsadj, nfadj, ta, tb, b2a.reshape(1, -1), b2b.reshape(1, -1), Wp)

    return (p1, p2, x1, x2)
